# Initial kernel scaffold; baseline (speedup 1.0000x reference)
import jax
import jax.numpy as jnp
from jax.experimental import pallas as pl


def kernel(x, t, pos, edge_index_local, edge_index_global, edge_attr_global, batch, batch_edge_global, W_time, b_time, W_atom, b_atom, W_at, b_at, W_sh, b_sh, W_b0, b_b0, W_b1, b_b1, W_co, W_al, b_al):
    raise NotImplementedError("write your pallas kernel here")


# speedup vs baseline: 2.4827x; 2.4827x over previous
import functools

import jax
import jax.numpy as jnp
from jax import lax
from jax.experimental import pallas as pl
from jax.experimental.pallas import tpu as pltpu
from jax.experimental.pallas import tpu_sc as plsc

N = 10000
G = 256
EG = 320000
SDIM = 256
NA = 16
NB = 5

NBLK = 256
NPAD = 10240
NNB = NPAD // NBLK

NC = 2
NS = 16
NW = NC * NS
EB = 128
EPW = 10240
EGP = NW * EPW
ECH = EPW // EB

EBLK = 2048
NEB = EGP // EBLK


def _mean_body(pos_ref, batch_ref, mean_ref, acc_ref):
    i = pl.program_id(0)
    bcol = batch_ref[...]
    gids = lax.broadcasted_iota(jnp.int32, (NBLK, G), 1)
    onehot = (bcol == gids).astype(jnp.float32)
    ext = jnp.concatenate(
        [pos_ref[...], jnp.ones((NBLK, 1), jnp.float32)], axis=1)
    contrib = lax.dot_general(onehot, ext, (((0,), (0,)), ((), ())),
                              preferred_element_type=jnp.float32)

    @pl.when(i == 0)
    def _():
        acc_ref[...] = contrib

    @pl.when(i > 0)
    def _():
        acc_ref[...] = acc_ref[...] + contrib

    @pl.when(i == NNB - 1)
    def _():
        acc = acc_ref[...]
        mean_ref[...] = acc / jnp.maximum(acc[:, 3:4], 1.0)


def _graph_mean(pos_pad, batch_col):
    return pl.pallas_call(
        _mean_body,
        grid=(NNB,),
        in_specs=[
            pl.BlockSpec((NBLK, 3), lambda i: (i, 0)),
            pl.BlockSpec((NBLK, 1), lambda i: (i, 0)),
        ],
        out_specs=pl.BlockSpec((G, 4), lambda i: (0, 0)),
        out_shape=jax.ShapeDtypeStruct((G, 4), jnp.float32),
        scratch_shapes=[pltpu.VMEM((G, 4), jnp.float32)],
    )(pos_pad, batch_col)


def _node_body(pos_ref, batch_ref, x_ref, mean_ref, t_ref,
               wti_ref, bti_ref, wat_ref, bat_ref, wa2_ref, ba2_ref,
               wsh_ref, bsh_ref, wb0_ref, wal_ref, bal_ref,
               posc_ref, u_ref, at_ref):
    bcol = batch_ref[...]
    gids = lax.broadcasted_iota(jnp.int32, (NBLK, G), 1)
    onehot = (bcol == gids).astype(jnp.float32)

    mean3 = mean_ref[...][:, 0:3]
    posc = pos_ref[...] - jnp.dot(onehot, mean3,
                                  preferred_element_type=jnp.float32)
    posc_ref[...] = posc

    temb = t_ref[...] * wti_ref[...] + bti_ref[...]
    tnode = jnp.dot(onehot, temb, preferred_element_type=jnp.float32)

    s = jnp.dot(x_ref[...], wat_ref[...],
                preferred_element_type=jnp.float32) + bat_ref[...]
    s = jnp.dot(s + tnode, wa2_ref[...],
                preferred_element_type=jnp.float32) + ba2_ref[...]
    sh = jax.nn.silu(jnp.dot(s, wsh_ref[...],
                             preferred_element_type=jnp.float32) + bsh_ref[...])

    u_ref[...] = jnp.dot(sh, wb0_ref[...], preferred_element_type=jnp.float32)
    at_ref[...] = jnp.dot(sh, wal_ref[...],
                          preferred_element_type=jnp.float32) + bal_ref[...]


def _node_dense(pos_pad, batch_col, x_pad, mean4, t,
                W_time, b_time, W_atom, b_atom, W_at, b_at,
                W_sh, b_sh, W_b0c, W_al, b_al):
    full = lambda r, c: pl.BlockSpec((r, c), lambda i: (0, 0))
    return pl.pallas_call(
        _node_body,
        grid=(NNB,),
        in_specs=[
            pl.BlockSpec((NBLK, 3), lambda i: (i, 0)),
            pl.BlockSpec((NBLK, 1), lambda i: (i, 0)),
            pl.BlockSpec((NBLK, NA), lambda i: (i, 0)),
            full(G, 4), full(G, 1),
            full(1, SDIM), full(1, SDIM),
            full(NA, SDIM), full(1, SDIM),
            full(SDIM, SDIM), full(1, SDIM),
            full(SDIM, SDIM), full(1, SDIM),
            full(SDIM, SDIM),
            full(SDIM, 2 * NA), full(1, 2 * NA),
        ],
        out_specs=[
            pl.BlockSpec((NBLK, 3), lambda i: (i, 0)),
            pl.BlockSpec((NBLK, SDIM), lambda i: (i, 0)),
            pl.BlockSpec((NBLK, 2 * NA), lambda i: (i, 0)),
        ],
        out_shape=[
            jax.ShapeDtypeStruct((NPAD, 3), jnp.float32),
            jax.ShapeDtypeStruct((NPAD, SDIM), jnp.float32),
            jax.ShapeDtypeStruct((NPAD, 2 * NA), jnp.float32),
        ],
    )(pos_pad, batch_col, x_pad, mean4, t,
      W_time.reshape(1, SDIM), b_time.reshape(1, SDIM),
      W_atom, b_atom.reshape(1, SDIM), W_at, b_at.reshape(1, SDIM),
      W_sh, b_sh.reshape(1, SDIM), W_b0c, W_al, b_al.reshape(1, 2 * NA))


def _edge_sc_body(src_hbm, dst_hbm, posq_hbm, u_hbm,
                  fsum_hbm, pdiff_hbm,
                  idxs_v, idxd_v, rows_s, rows_d,
                  pos_s, pos_d, pdiff_v, sem0, sem1):
    wid = lax.axis_index("s") * NC + lax.axis_index("c")
    base0 = wid * EPW

    def chunk(tc, carry):
        base = base0 + tc * EB
        pltpu.sync_copy(src_hbm.at[pl.ds(base, EB)], idxs_v)
        pltpu.sync_copy(dst_hbm.at[pl.ds(base, EB)], idxd_v)
        cs = pltpu.async_copy(u_hbm.at[idxs_v], rows_s, sem0)
        cd = pltpu.async_copy(u_hbm.at[idxd_v], rows_d, sem0)
        ps = pltpu.async_copy(posq_hbm.at[idxs_v], pos_s, sem1)
        pd = pltpu.async_copy(posq_hbm.at[idxd_v], pos_d, sem1)
        cs.wait()
        cd.wait()
        ps.wait()
        pd.wait()

        def add_row(e, c2):
            for k in range(SDIM // 16):
                ksl = pl.ds(k * 16, 16)
                rows_s[e, ksl] = rows_s[e, ksl] + rows_d[e, ksl]
            psl = pl.ds(0, 16)
            diff = pos_s[e, psl] - pos_d[e, psl]
            pdiff_v[e, psl] = diff
            return c2

        lax.fori_loop(0, EB, add_row, 0)
        pltpu.sync_copy(rows_s, fsum_hbm.at[pl.ds(base, EB)])
        pltpu.sync_copy(pdiff_v, pdiff_hbm.at[pl.ds(base, EB)])
        return carry

    lax.fori_loop(0, ECH, chunk, 0)


def _edge_gather(src_idx, dst_idx, posq, u):
    mesh = plsc.VectorSubcoreMesh(core_axis_name="c", subcore_axis_name="s")
    f = functools.partial(
        pl.kernel, _edge_sc_body, mesh=mesh,
        out_type=[
            jax.ShapeDtypeStruct((EGP, SDIM), jnp.float32),
            jax.ShapeDtypeStruct((EGP, 16), jnp.float32),
        ],
        scratch_types=[
            pltpu.VMEM((EB,), jnp.int32),
            pltpu.VMEM((EB,), jnp.int32),
            pltpu.VMEM((EB, SDIM), jnp.float32),
            pltpu.VMEM((EB, SDIM), jnp.float32),
            pltpu.VMEM((EB, 128), jnp.float32),
            pltpu.VMEM((EB, 128), jnp.float32),
            pltpu.VMEM((EB, 16), jnp.float32),
            pltpu.SemaphoreType.DMA,
            pltpu.SemaphoreType.DMA,
        ],
    )()
    return f(src_idx, dst_idx, posq, u)


def _bond_body(fsum_ref, pdiff_ref, wd_ref, bb0_ref, wb1_ref, bb1_ref,
               out_ref):
    pd = pdiff_ref[...]
    d = jnp.sqrt(jnp.sum(pd * pd, axis=1, keepdims=True))
    h = fsum_ref[...] + d * wd_ref[...] + bb0_ref[...]
    hs = jax.nn.silu(h)
    out_ref[...] = jnp.dot(hs, wb1_ref[...],
                           preferred_element_type=jnp.float32) + bb1_ref[...]


def _bond_tail(fsum, pdiff, wd, bb0, wb1p, bb1p):
    full = lambda r, c: pl.BlockSpec((r, c), lambda i: (0, 0))
    return pl.pallas_call(
        _bond_body,
        grid=(NEB,),
        in_specs=[
            pl.BlockSpec((EBLK, SDIM), lambda i: (i, 0)),
            pl.BlockSpec((EBLK, 16), lambda i: (i, 0)),
            full(1, SDIM), full(1, SDIM),
            full(SDIM, 16), full(1, 16),
        ],
        out_specs=pl.BlockSpec((EBLK, 16), lambda i: (i, 0)),
        out_shape=jax.ShapeDtypeStruct((EGP, 16), jnp.float32),
    )(fsum, pdiff, wd, bb0, wb1p, bb1p)


def kernel(x, t, pos, edge_index_local, edge_index_global, edge_attr_global,
           batch, batch_edge_global, W_time, b_time, W_atom, b_atom, W_at,
           b_at, W_sh, b_sh, W_b0, b_b0, W_b1, b_b1, W_co, W_al, b_al):
    pos_pad = jnp.pad(pos, ((0, NPAD - N), (0, 0)))
    x_pad = jnp.pad(x, ((0, NPAD - N), (0, 0)))
    batch_col = jnp.pad(batch, (0, NPAD - N),
                        constant_values=G).reshape(NPAD, 1)

    mean4 = _graph_mean(pos_pad, batch_col)
    posc_pad, u, at = _node_dense(
        pos_pad, batch_col, x_pad, mean4, t,
        W_time, b_time, W_atom, b_atom, W_at, b_at,
        W_sh, b_sh, W_b0[:SDIM, :], W_al, b_al)

    src_idx = jnp.pad(edge_index_global[0], (0, EGP - EG))
    dst_idx = jnp.pad(edge_index_global[1], (0, EGP - EG))
    posq = jnp.pad(posc_pad, ((0, 0), (0, 125)))

    fsum, pdiff = _edge_gather(src_idx, dst_idx, posq, u)

    bonds = _bond_tail(fsum, pdiff,
                       W_b0[SDIM:SDIM + 1, :], b_b0.reshape(1, SDIM),
                       jnp.pad(W_b1, ((0, 0), (0, 16 - 2 * NB))),
                       jnp.pad(b_b1, (0, 16 - 2 * NB)).reshape(1, 16))

    pos_c = posc_pad[:N]
    coords_pred = pos_c
    coords_eps = jnp.zeros((N, 3), jnp.float32)
    atoms_eps = at[:N, :NA]
    atoms_pred = at[:N, NA:]
    bonds_pred = bonds[:EG, :NB]
    bonds_eps = bonds[:EG, NB:2 * NB]
    return (coords_pred, coords_eps, atoms_pred, atoms_eps,
            bonds_pred, bonds_eps, pos_c, x, edge_attr_global)


# speedup vs baseline: 3.1641x; 1.2745x over previous
import functools

import jax
import jax.numpy as jnp
from jax import lax
from jax.experimental import pallas as pl
from jax.experimental.pallas import tpu as pltpu
from jax.experimental.pallas import tpu_sc as plsc

N = 10000
G = 256
EG = 320000
SDIM = 256
NA = 16
NB = 5

NBLK = 256
NPAD = 10240
NNB = NPAD // NBLK

NC = 2
NS = 16
NW = NC * NS
EB = 64
EPW = 10240
EGP = NW * EPW
ECH = EPW // EB
NPAIR = ECH // 2
TBW = 384

EBLK = 2048
NEB = EGP // EBLK


def _mean_body(pos_ref, batch_ref, mean_ref, acc_ref):
    i = pl.program_id(0)
    bcol = batch_ref[...]
    gids = lax.broadcasted_iota(jnp.int32, (NBLK, G), 1)
    onehot = (bcol == gids).astype(jnp.float32)
    ext = jnp.concatenate(
        [pos_ref[...], jnp.ones((NBLK, 1), jnp.float32)], axis=1)
    contrib = lax.dot_general(onehot, ext, (((0,), (0,)), ((), ())),
                              preferred_element_type=jnp.float32)

    @pl.when(i == 0)
    def _():
        acc_ref[...] = contrib

    @pl.when(i > 0)
    def _():
        acc_ref[...] = acc_ref[...] + contrib

    @pl.when(i == NNB - 1)
    def _():
        acc = acc_ref[...]
        mean_ref[...] = acc / jnp.maximum(acc[:, 3:4], 1.0)


def _graph_mean(pos_pad, batch_col):
    return pl.pallas_call(
        _mean_body,
        grid=(NNB,),
        in_specs=[
            pl.BlockSpec((NBLK, 3), lambda i: (i, 0)),
            pl.BlockSpec((NBLK, 1), lambda i: (i, 0)),
        ],
        out_specs=pl.BlockSpec((G, 4), lambda i: (0, 0)),
        out_shape=jax.ShapeDtypeStruct((G, 4), jnp.float32),
        scratch_shapes=[pltpu.VMEM((G, 4), jnp.float32)],
    )(pos_pad, batch_col)


def _node_body(pos_ref, batch_ref, x_ref, mean_ref, t_ref,
               wti_ref, bti_ref, wat_ref, bat_ref, wa2_ref, ba2_ref,
               wsh_ref, bsh_ref, wb0_ref, wal_ref, bal_ref,
               posc_ref, tab_ref, at_ref):
    bcol = batch_ref[...]
    gids = lax.broadcasted_iota(jnp.int32, (NBLK, G), 1)
    onehot = (bcol == gids).astype(jnp.float32)

    mean3 = mean_ref[...][:, 0:3]
    posc = pos_ref[...] - jnp.dot(onehot, mean3,
                                  preferred_element_type=jnp.float32)
    posc_ref[...] = posc

    temb = t_ref[...] * wti_ref[...] + bti_ref[...]
    tnode = jnp.dot(onehot, temb, preferred_element_type=jnp.float32)

    s = jnp.dot(x_ref[...], wat_ref[...],
                preferred_element_type=jnp.float32) + bat_ref[...]
    s = jnp.dot(s + tnode, wa2_ref[...],
                preferred_element_type=jnp.float32) + ba2_ref[...]
    sh = jax.nn.silu(jnp.dot(s, wsh_ref[...],
                             preferred_element_type=jnp.float32) + bsh_ref[...])

    tab_ref[:, 0:SDIM] = jnp.dot(sh, wb0_ref[...],
                                 preferred_element_type=jnp.float32)
    tab_ref[:, SDIM:TBW] = jnp.concatenate(
        [posc, jnp.zeros((NBLK, TBW - SDIM - 3), jnp.float32)], axis=1)
    at_ref[...] = jnp.dot(sh, wal_ref[...],
                          preferred_element_type=jnp.float32) + bal_ref[...]


def _node_dense(pos_pad, batch_col, x_pad, mean4, t,
                W_time, b_time, W_atom, b_atom, W_at, b_at,
                W_sh, b_sh, W_b0c, W_al, b_al):
    full = lambda r, c: pl.BlockSpec((r, c), lambda i: (0, 0))
    return pl.pallas_call(
        _node_body,
        grid=(NNB,),
        in_specs=[
            pl.BlockSpec((NBLK, 3), lambda i: (i, 0)),
            pl.BlockSpec((NBLK, 1), lambda i: (i, 0)),
            pl.BlockSpec((NBLK, NA), lambda i: (i, 0)),
            full(G, 4), full(G, 1),
            full(1, SDIM), full(1, SDIM),
            full(NA, SDIM), full(1, SDIM),
            full(SDIM, SDIM), full(1, SDIM),
            full(SDIM, SDIM), full(1, SDIM),
            full(SDIM, SDIM),
            full(SDIM, 2 * NA), full(1, 2 * NA),
        ],
        out_specs=[
            pl.BlockSpec((NBLK, 3), lambda i: (i, 0)),
            pl.BlockSpec((NBLK, TBW), lambda i: (i, 0)),
            pl.BlockSpec((NBLK, 2 * NA), lambda i: (i, 0)),
        ],
        out_shape=[
            jax.ShapeDtypeStruct((NPAD, 3), jnp.float32),
            jax.ShapeDtypeStruct((NPAD, TBW), jnp.float32),
            jax.ShapeDtypeStruct((NPAD, 2 * NA), jnp.float32),
        ],
    )(pos_pad, batch_col, x_pad, mean4, t,
      W_time.reshape(1, SDIM), b_time.reshape(1, SDIM),
      W_atom, b_atom.reshape(1, SDIM), W_at, b_at.reshape(1, SDIM),
      W_sh, b_sh.reshape(1, SDIM), W_b0c, W_al, b_al.reshape(1, 2 * NA))


GBYTES = 2 * EB * TBW * 4
WBYTES = EB * TBW * 4


def _edge_sc_body(cidx_hbm, tab_hbm, out_hbm,
                  idx_v, rows_a, rows_b, semga, semgb, semwa, semwb):
    wid = lax.axis_index("s") * NC + lax.axis_index("c")
    pltpu.sync_copy(cidx_hbm.at[pl.ds(wid * 2 * EPW, 2 * EPW)], idx_v)
    ebase = wid * EPW

    def fire_gather(t, buf, sem):
        pltpu.async_copy(tab_hbm.at[idx_v.at[pl.ds(t * 2 * EB, 2 * EB)]],
                         buf, sem)

    def wait_gather(buf, sem):
        pltpu.make_async_copy(
            tab_hbm.at[idx_v.at[pl.ds(0, 2 * EB)]], buf, sem).wait()

    def wait_write(buf, sem):
        pltpu.make_async_copy(
            buf.at[pl.ds(0, EB)], out_hbm.at[pl.ds(ebase, EB)], sem).wait()

    def compute(buf):
        @plsc.parallel_loop(0, EB, unroll=2)
        def _(e):
            for k in range(SDIM // 16):
                sl = pl.ds(k * 16, 16)
                buf[e, sl] = buf[e, sl] + buf[EB + e, sl]
            psl = pl.ds(SDIM, 16)
            buf[e, psl] = buf[e, psl] - buf[EB + e, psl]

    def write_out(t, buf, sem):
        pltpu.async_copy(buf.at[pl.ds(0, EB)],
                         out_hbm.at[pl.ds(ebase + t * EB, EB)], sem)

    fire_gather(0, rows_a, semga)

    def pair(tt, carry):
        t0 = 2 * tt
        t1 = t0 + 1

        @pl.when(tt > 0)
        def _():
            wait_write(rows_b, semwb)
        fire_gather(t1, rows_b, semgb)
        wait_gather(rows_a, semga)
        compute(rows_a)
        write_out(t0, rows_a, semwa)
        wait_gather(rows_b, semgb)
        wait_write(rows_a, semwa)

        @pl.when(tt < NPAIR - 1)
        def _():
            fire_gather(t0 + 2, rows_a, semga)
        compute(rows_b)
        write_out(t1, rows_b, semwb)
        return carry

    lax.fori_loop(0, NPAIR, pair, 0)
    wait_write(rows_b, semwb)


def _edge_gather(cidx, tab):
    mesh = plsc.VectorSubcoreMesh(core_axis_name="c", subcore_axis_name="s")
    f = functools.partial(
        pl.kernel, _edge_sc_body, mesh=mesh,
        out_type=jax.ShapeDtypeStruct((EGP, TBW), jnp.float32),
        scratch_types=[
            pltpu.VMEM((2 * EPW,), jnp.int32),
            pltpu.VMEM((2 * EB, TBW), jnp.float32),
            pltpu.VMEM((2 * EB, TBW), jnp.float32),
            pltpu.SemaphoreType.DMA,
            pltpu.SemaphoreType.DMA,
            pltpu.SemaphoreType.DMA,
            pltpu.SemaphoreType.DMA,
        ],
    )()
    return f(cidx, tab)


def _bond_body(wide_ref, wd_ref, bb0_ref, wb1_ref, bb1_ref, out_ref):
    blk = wide_ref[...]
    fsum = blk[:, 0:SDIM]
    pd = blk[:, SDIM:SDIM + 16]
    d = jnp.sqrt(jnp.sum(pd * pd, axis=1, keepdims=True))
    h = fsum + d * wd_ref[...] + bb0_ref[...]
    hs = jax.nn.silu(h)
    out_ref[...] = jnp.dot(hs, wb1_ref[...],
                           preferred_element_type=jnp.float32) + bb1_ref[...]


def _bond_tail(wide, wd, bb0, wb1p, bb1p):
    full = lambda r, c: pl.BlockSpec((r, c), lambda i: (0, 0))
    return pl.pallas_call(
        _bond_body,
        grid=(NEB,),
        in_specs=[
            pl.BlockSpec((EBLK, TBW), lambda i: (i, 0)),
            full(1, SDIM), full(1, SDIM),
            full(SDIM, 16), full(1, 16),
        ],
        out_specs=pl.BlockSpec((EBLK, 16), lambda i: (i, 0)),
        out_shape=jax.ShapeDtypeStruct((EGP, 16), jnp.float32),
    )(wide, wd, bb0, wb1p, bb1p)


def kernel(x, t, pos, edge_index_local, edge_index_global, edge_attr_global,
           batch, batch_edge_global, W_time, b_time, W_atom, b_atom, W_at,
           b_at, W_sh, b_sh, W_b0, b_b0, W_b1, b_b1, W_co, W_al, b_al):
    pos_pad = jnp.pad(pos, ((0, NPAD - N), (0, 0)))
    x_pad = jnp.pad(x, ((0, NPAD - N), (0, 0)))
    batch_col = jnp.pad(batch, (0, NPAD - N),
                        constant_values=G).reshape(NPAD, 1)

    mean4 = _graph_mean(pos_pad, batch_col)
    posc_pad, tab, at = _node_dense(
        pos_pad, batch_col, x_pad, mean4, t,
        W_time, b_time, W_atom, b_atom, W_at, b_at,
        W_sh, b_sh, W_b0[:SDIM, :], W_al, b_al)

    src_idx = jnp.pad(edge_index_global[0], (0, EGP - EG))
    dst_idx = jnp.pad(edge_index_global[1], (0, EGP - EG))
    cht = EGP // EB
    cidx = jnp.stack([src_idx.reshape(cht, EB),
                      dst_idx.reshape(cht, EB)], axis=1).reshape(2 * EGP)

    wide = _edge_gather(cidx, tab)

    bonds = _bond_tail(wide,
                       W_b0[SDIM:SDIM + 1, :], b_b0.reshape(1, SDIM),
                       jnp.pad(W_b1, ((0, 0), (0, 16 - 2 * NB))),
                       jnp.pad(b_b1, (0, 16 - 2 * NB)).reshape(1, 16))

    pos_c = posc_pad[:N]
    coords_pred = pos_c
    coords_eps = jnp.zeros((N, 3), jnp.float32)
    atoms_eps = at[:N, :NA]
    atoms_pred = at[:N, NA:]
    bonds_pred = bonds[:EG, :NB]
    bonds_eps = bonds[:EG, NB:2 * NB]
    return (coords_pred, coords_eps, atoms_pred, atoms_eps,
            bonds_pred, bonds_eps, pos_c, x, edge_attr_global)
